# Initial kernel scaffold; baseline (speedup 1.0000x reference)
import jax
import jax.numpy as jnp
from jax.experimental import pallas as pl


def kernel(x, edge_index, batch, W1, b1, W2, b2, W3, b3, W4, b4, g1, be1, g2, be2, g3, be3, g4, be4, Wf1, bf1, Wf2, bf2, Wf3, bf3):
    raise NotImplementedError("write your pallas kernel here")


# speedup vs baseline: 10.4595x; 10.4595x over previous
import functools

import jax
import jax.numpy as jnp
from jax import lax
from jax.experimental import pallas as pl
from jax.experimental.pallas import tpu as pltpu
from jax.experimental.pallas import tpu_sc as plsc

N = 50000
NP = 50048
E = 800000
G = 64
NTILES = 16
NCORES = 2
K = 1000
RPT = NP // NTILES
ZR = 184


def _fill_const(ref, rows, width, value):
    def body(i, carry):
        for half in range(width // 16):
            ref[i, pl.ds(half * 16, 16)] = jnp.full((16,), value, jnp.float32)
        return carry
    lax.fori_loop(0, rows, body, None)


def _make_deg_kernel():
    W = 16
    EPT = E // (NCORES * NTILES)
    mesh = plsc.VectorSubcoreMesh(core_axis_name="c", subcore_axis_name="s")

    @functools.partial(
        pl.kernel,
        out_type=jax.ShapeDtypeStruct((2 * NP, W), jnp.float32),
        mesh=mesh,
        compiler_params=pltpu.CompilerParams(use_tc_tiling_on_sc=False),
        scratch_types=[
            pltpu.VMEM((K,), jnp.int32),
            pltpu.VMEM((K, W), jnp.float32),
            pltpu.VMEM((ZR, W), jnp.float32),
            pltpu.VMEM_SHARED((NP, W), jnp.float32),
        ],
    )
    def deg_kernel(dst_hbm, out_hbm, dstv, ones_v, zero_v, acc):
        core = lax.axis_index("c")
        sub = lax.axis_index("s")
        _fill_const(ones_v, K, W, 1.0)
        _fill_const(zero_v, ZR, W, 0.0)
        r0 = sub * RPT
        for r in range(RPT // ZR):
            pltpu.sync_copy(zero_v, acc.at[pl.ds(r0 + r * ZR, ZR)])
        plsc.subcore_barrier()
        e0 = (core * NTILES + sub) * EPT

        def win(w, carry):
            base = e0 + w * K
            pltpu.sync_copy(dst_hbm.at[pl.ds(base, K)], dstv)
            pltpu.sync_copy(ones_v, acc.at[dstv], add=True)
            return carry

        lax.fori_loop(0, EPT // K, win, None)
        plsc.subcore_barrier()
        pltpu.sync_copy(acc.at[pl.ds(r0, RPT)],
                        out_hbm.at[pl.ds(core * NP + r0, RPT)])

    return deg_kernel


def _make_agg_kernel(C, W, edge_split):
    EPT = E // (NCORES * NTILES) if edge_split else E // NTILES
    out_rows = 2 * NP if edge_split else NP
    mesh = plsc.VectorSubcoreMesh(core_axis_name="c", subcore_axis_name="s")
    out_types = [jax.ShapeDtypeStruct((out_rows, W), jnp.float32)
                 for _ in range(C)]

    @functools.partial(
        pl.kernel,
        out_type=out_types,
        mesh=mesh,
        compiler_params=pltpu.CompilerParams(use_tc_tiling_on_sc=False),
        scratch_types=[
            pltpu.VMEM((K,), jnp.int32),
            pltpu.VMEM((K,), jnp.int32),
            pltpu.VMEM((K, W), jnp.float32),
            pltpu.VMEM((ZR, W), jnp.float32),
            pltpu.VMEM_SHARED((NP, W), jnp.float32),
            pltpu.SemaphoreType.DMA,
        ],
    )
    def agg_kernel(src_hbm, dst_hbm, *refs):
        u_refs = refs[:C]
        out_refs = refs[C:2 * C]
        srcv, dstv, msg, zero_v, acc, sem = refs[2 * C:]
        core = lax.axis_index("c")
        sub = lax.axis_index("s")
        _fill_const(zero_v, ZR, W, 0.0)
        r0 = sub * RPT

        def process(c):
            for r in range(RPT // ZR):
                pltpu.sync_copy(zero_v, acc.at[pl.ds(r0 + r * ZR, ZR)])
            plsc.subcore_barrier()
            if edge_split:
                e0 = (core * NTILES + sub) * EPT
            else:
                e0 = sub * EPT

            def win(w, carry):
                base = e0 + w * K
                pltpu.sync_copy(src_hbm.at[pl.ds(base, K)], srcv)
                pltpu.async_copy(u_refs[c].at[srcv], msg, sem).wait()
                pltpu.sync_copy(dst_hbm.at[pl.ds(base, K)], dstv)
                pltpu.sync_copy(msg, acc.at[dstv], add=True)
                return carry

            lax.fori_loop(0, EPT // K, win, None)
            plsc.subcore_barrier()
            if edge_split:
                o0 = core * NP + r0
            else:
                o0 = r0
            pltpu.sync_copy(acc.at[pl.ds(r0, RPT)],
                            out_refs[c].at[pl.ds(o0, RPT)])

        for c in range(C):
            if edge_split:
                process(c)
            else:
                @pl.when(core == (c % NCORES))
                def _(c=c):
                    process(c)

    return agg_kernel


def _lazy(factory):
    cache = {}

    def call(*args):
        if "k" not in cache:
            cache["k"] = factory()
        return cache["k"](*args)

    return call


_deg = _lazy(_make_deg_kernel)
_agg1 = _lazy(lambda: _make_agg_kernel(1, 16, True))
_agg2 = _lazy(lambda: _make_agg_kernel(4, 16, False))
_agg4 = _lazy(lambda: _make_agg_kernel(8, 16, False))


BN_ROWS = 1472
NSTEPS = NP // BN_ROWS


def _row_spec(width, block_off=0):
    return pl.BlockSpec((BN_ROWS, width), lambda i: (i + block_off, 0))


def _full_spec(shape):
    return pl.BlockSpec(shape, lambda i: tuple(0 for _ in shape))


def _rowmask(i):
    rid = i * BN_ROWS + lax.broadcasted_iota(jnp.int32, (BN_ROWS, 1), 0)
    return (rid < N).astype(jnp.float32)


def _prep_call(degp, x16):
    def body(d0, d1, x, dis_o, u1_o):
        d = d0[:, 0:1] + d1[:, 0:1] + 1.0
        dis = lax.rsqrt(d)
        dis_o[...] = dis
        u1_o[...] = dis * x[...]

    return pl.pallas_call(
        body,
        grid=(NSTEPS,),
        in_specs=[_row_spec(16), _row_spec(16, NSTEPS), _row_spec(16)],
        out_specs=[_row_spec(1), _row_spec(16)],
        out_shape=[jax.ShapeDtypeStruct((NP, 1), jnp.float32),
                   jax.ShapeDtypeStruct((NP, 16), jnp.float32)],
    )(degp, degp, x16)


def _layer_a_call(agg_parts, u_parts, dis, Wmat, brow, split_agg=False):
    C = len(u_parts)
    cw = u_parts[0].shape[1]
    d_in, d_out = Wmat.shape
    nagg = 2 * C if split_agg else C

    def body(*refs):
        aggs = refs[:nagg]
        us = refs[nagg:nagg + C]
        dis_r, w_r, b_r = refs[nagg + C:nagg + C + 3]
        z_r, stats_r = refs[nagg + C + 3:nagg + C + 5]
        sacc = refs[nagg + C + 5]
        i = pl.program_id(0)
        if split_agg:
            parts = [aggs[c][...] + aggs[C + c][...] + us[c][...]
                     for c in range(C)]
        else:
            parts = [aggs[c][...] + us[c][...] for c in range(C)]
        pre = parts[0] if C == 1 else jnp.concatenate(parts, axis=1)
        pre = dis_r[...] * pre
        z = jnp.dot(pre, w_r[...], preferred_element_type=jnp.float32) + b_r[...]
        z_r[...] = z

        @pl.when(i == 0)
        def _():
            sacc[...] = jnp.zeros_like(sacc)

        zm = _rowmask(i) * z
        sacc[0:1, :] += jnp.sum(zm, axis=0, keepdims=True)
        sacc[1:2, :] += jnp.sum(zm * z, axis=0, keepdims=True)

        @pl.when(i == NSTEPS - 1)
        def _():
            stats_r[...] = sacc[...]

    if split_agg:
        agg_args = [agg_parts[0], agg_parts[0]]
        agg_specs = [_row_spec(cw), _row_spec(cw, NSTEPS)]
    else:
        agg_args = list(agg_parts)
        agg_specs = [_row_spec(cw)] * C

    return pl.pallas_call(
        body,
        grid=(NSTEPS,),
        in_specs=(agg_specs + [_row_spec(cw)] * C
                  + [_row_spec(1), _full_spec((d_in, d_out)),
                     _full_spec((1, d_out))]),
        out_specs=[_row_spec(d_out), _full_spec((2, d_out))],
        out_shape=[jax.ShapeDtypeStruct((NP, d_out), jnp.float32),
                   jax.ShapeDtypeStruct((2, d_out), jnp.float32)],
        scratch_shapes=[pltpu.VMEM((2, d_out), jnp.float32)],
    )(*agg_args, *u_parts, dis, Wmat, brow)


def _layer_b_call(z, stats, grow, berow, dis, scale_by_dis):
    d = z.shape[1]
    C = d // 16

    def body(*refs):
        z_r, st_r, g_r, be_r, dis_r = refs[:5]
        outs = refs[5:]
        m = st_r[0:1, :] * (1.0 / N)
        var = st_r[1:2, :] * (1.0 / N) - m * m
        scale = g_r[...] * lax.rsqrt(var + 1e-5)
        shift = be_r[...] - m * scale
        h = jnp.maximum(z_r[...] * scale + shift, 0.0)
        if scale_by_dis:
            h = dis_r[...] * h
        for c in range(C):
            outs[c][...] = h[:, c * 16:(c + 1) * 16]

    return pl.pallas_call(
        body,
        grid=(NSTEPS,),
        in_specs=[_row_spec(d), _full_spec((2, d)), _full_spec((1, d)),
                  _full_spec((1, d)), _row_spec(1)],
        out_specs=[_row_spec(16)] * C,
        out_shape=[jax.ShapeDtypeStruct((NP, 16), jnp.float32)
                   for _ in range(C)],
    )(z, stats, grow, berow, dis)


def _pool_head_call(h_parts, batch2d, Wf1, bf1, Wf2, bf2, Wf3, bf3):
    CP = len(h_parts)

    def body(*refs):
        hs = refs[:CP]
        b_r = refs[CP]
        w1, b1, w2, b2, w3, b3 = refs[CP + 1:CP + 7]
        out_r = refs[CP + 7]
        pooled_s, cnt_s = refs[CP + 8:CP + 10]
        i = pl.program_id(0)

        @pl.when(i == 0)
        def _():
            pooled_s[...] = jnp.zeros_like(pooled_s)
            cnt_s[...] = jnp.zeros_like(cnt_s)

        gid = lax.broadcasted_iota(jnp.int32, (1, G), 1)
        mask = (b_r[...] == gid).astype(jnp.float32)
        h = jnp.concatenate([hs[c][...] for c in range(CP)], axis=1)
        pooled_s[...] += lax.dot_general(
            mask, h, (((0,), (0,)), ((), ())),
            preferred_element_type=jnp.float32)
        cnt_s[...] += lax.dot_general(
            mask, jnp.ones((BN_ROWS, 1), jnp.float32),
            (((0,), (0,)), ((), ())),
            preferred_element_type=jnp.float32)

        @pl.when(i == NSTEPS - 1)
        def _():
            pm = pooled_s[...] / jnp.maximum(cnt_s[...], 1.0)
            o = jnp.maximum(
                jnp.dot(pm, w1[...], preferred_element_type=jnp.float32)
                + b1[...], 0.0)
            o = jnp.maximum(
                jnp.dot(o, w2[...], preferred_element_type=jnp.float32)
                + b2[...], 0.0)
            out_r[...] = (jnp.dot(o, w3[...],
                                  preferred_element_type=jnp.float32)
                          + b3[...])

    return pl.pallas_call(
        body,
        grid=(NSTEPS,),
        in_specs=([_row_spec(16)] * CP
                  + [_row_spec(1), _full_spec((256, 128)), _full_spec((1, 128)),
                     _full_spec((128, 64)), _full_spec((1, 64)),
                     _full_spec((64, 1)), _full_spec((1, 1))]),
        out_specs=_full_spec((G, 1)),
        out_shape=jax.ShapeDtypeStruct((G, 1), jnp.float32),
        scratch_shapes=[pltpu.VMEM((G, 256), jnp.float32),
                        pltpu.VMEM((G, 1), jnp.float32)],
    )(*h_parts, batch2d, Wf1, bf1, Wf2, bf2, Wf3, bf3)


def kernel(x, edge_index, batch, W1, b1, W2, b2, W3, b3, W4, b4,
           g1, be1, g2, be2, g3, be3, g4, be4,
           Wf1, bf1, Wf2, bf2, Wf3, bf3):
    src = edge_index[0].astype(jnp.int32)
    dst = edge_index[1].astype(jnp.int32)
    x16 = jnp.pad(x, ((0, NP - N), (0, 13)))
    W1p = jnp.pad(W1, ((0, 13), (0, 0)))
    row = lambda v: v.reshape(1, -1)

    degp = _deg(dst)
    dis, u1 = _prep_call(degp, x16)

    a1p = _agg1(src, dst, u1)
    z1, s1 = _layer_a_call(a1p, [u1], dis, W1p, row(b1), split_agg=True)
    u2 = _layer_b_call(z1, s1, row(g1), row(be1), dis, True)

    a2 = _agg2(src, dst, *u2)
    z2, s2 = _layer_a_call(list(a2), u2, dis, W2, row(b2))
    u3 = _layer_b_call(z2, s2, row(g2), row(be2), dis, True)

    a3 = _agg4(src, dst, *u3)
    z3, s3 = _layer_a_call(list(a3), u3, dis, W3, row(b3))
    u4 = _layer_b_call(z3, s3, row(g3), row(be3), dis, True)

    a4 = _agg4(src, dst, *u4)
    z4, s4 = _layer_a_call(list(a4), u4, dis, W4, row(b4))
    h4 = _layer_b_call(z4, s4, row(g4), row(be4), dis, False)

    batch2d = jnp.pad(batch.astype(jnp.int32), (0, NP - N),
                      constant_values=G).reshape(NP, 1)
    return _pool_head_call(list(h4), batch2d, Wf1, row(bf1), Wf2, row(bf2),
                           Wf3, row(bf3))
